# Initial kernel scaffold; baseline (speedup 1.0000x reference)
import jax
import jax.numpy as jnp
from jax.experimental import pallas as pl


def kernel(x, seq_lengths, w1, b1, w2, b2):
    raise NotImplementedError("write your pallas kernel here")


# speedup vs baseline: 139.0095x; 139.0095x over previous
import functools

import jax
import jax.numpy as jnp
from jax import lax
from jax.experimental import pallas as pl
from jax.experimental.pallas import tpu as pltpu

_VMEM_LIMIT = 64 * 1024 * 1024

_KH1, _KW1, _SH1, _SW1, _PH1, _PW1 = 41, 11, 2, 2, 20, 5
_KH2, _KW2, _SH2, _SW2, _PH2, _PW2 = 21, 11, 2, 1, 10, 5


def _fused_body(seqs_ref, xe_ref, xo_ref, band1_ref, bias1_ref, band2_ref,
                bias2_ref, o_ref, pe_ref, po_ref, x2_ref, *,
                H, Wh, W1, W2, R1, R2):
    n = pl.program_id(0)
    s0 = seqs_ref[n, 0]
    s1 = seqs_ref[n, 1]

    xe = jnp.clip(xe_ref[0], -1.0, 1.0)
    xo = jnp.clip(xo_ref[0], -1.0, 1.0)
    col = lax.broadcasted_iota(jnp.int32, (H, Wh), 1)
    xe = jnp.where(2 * col < s0, xe, jnp.bfloat16(0))
    xo = jnp.where(2 * col + 1 < s0, xo, jnp.bfloat16(0))
    pe_ref[...] = jnp.zeros_like(pe_ref)
    po_ref[...] = jnp.zeros_like(po_ref)
    pe_ref[:, (_PW1 + 1) // 2:(_PW1 + 1) // 2 + Wh] = xo
    po_ref[:, (_PW1 - 1) // 2:(_PW1 - 1) // 2 + Wh] = xe

    acc1 = jnp.zeros((R1, W1), jnp.float32)
    for kx in range(_KW1):
        src = pe_ref if kx % 2 == 0 else po_ref
        j = kx // 2
        acc1 = acc1 + jnp.dot(band1_ref[kx], src[:, j:j + W1],
                              preferred_element_type=jnp.float32)
    y1 = jnp.clip(acc1 + bias1_ref[...], 0.0, 20.0)
    wcol1 = lax.broadcasted_iota(jnp.int32, (R1, W1), 1)
    y1 = jnp.where(wcol1 < s1, y1, 0.0)

    x2_ref[...] = jnp.zeros_like(x2_ref)
    x2_ref[:, _PW2:_PW2 + W1] = y1.astype(jnp.bfloat16)

    acc2 = jnp.zeros((R2, W2), jnp.float32)
    for kx in range(_KW2):
        acc2 = acc2 + jnp.dot(band2_ref[kx], x2_ref[:, kx:kx + W2],
                              preferred_element_type=jnp.float32)
    y2 = jnp.clip(acc2 + bias2_ref[...], 0.0, 10.0)
    wcol2 = lax.broadcasted_iota(jnp.int32, (R2, W2), 1)
    o_ref[0] = jnp.where(wcol2 < s1, y2, 0.0)


def _band_from_weights(w, hout, hin, stride, pad, dtype):
    co, ci, kh, kw = w.shape
    ho = jnp.arange(hout)
    ky = jnp.arange(kh)
    h = jnp.arange(hin)
    sel = (h[None, None, :] ==
           (stride * ho[:, None, None] + ky[None, :, None] - pad))
    sel = sel.astype(jnp.float32)
    band = jnp.einsum('oiyx,jyh->xojih', w, sel)
    return band.reshape(kw, co * hout, ci * hin).astype(dtype)


def kernel(x, seq_lengths, w1, b1, w2, b2):
    N, Cin, H, W = x.shape
    C1 = w1.shape[0]
    C2 = w2.shape[0]
    Hout1 = (H + 2 * _PH1 - _KH1) // _SH1 + 1
    W1out = (W + 2 * _PW1 - _KW1) // _SW1 + 1
    Hout2 = (Hout1 + 2 * _PH2 - _KH2) // _SH2 + 1
    W2out = (W1out + 2 * _PW2 - _KW2) // _SW2 + 1
    R1, R2 = C1 * Hout1, C2 * Hout2

    s0 = seq_lengths.astype(jnp.int32)
    s1 = (s0 + 2 * _PW1 - (_KW1 - 1) - 1) // _SW1 + 1
    s2 = (s1 + 2 * _PW2 - (_KW2 - 1) - 1) // _SW2 + 1
    seqs = jnp.stack([s0, s1, s2], axis=1)

    xsq = x[:, 0].astype(jnp.bfloat16)
    xe = xsq[:, :, 0::2]
    xo = xsq[:, :, 1::2]
    Wh = xe.shape[-1]
    PW = max((_PW1 + 1) // 2 + Wh, _KW1 // 2 + W1out)

    band1 = _band_from_weights(w1, Hout1, H, _SH1, _PH1, jnp.bfloat16)
    band2 = _band_from_weights(w2, Hout2, Hout1, _SH2, _PH2, jnp.bfloat16)
    bias1 = jnp.repeat(b1, Hout1).reshape(R1, 1)
    bias2 = jnp.repeat(b2, Hout2).reshape(R2, 1)

    body = functools.partial(_fused_body, H=H, Wh=Wh, W1=W1out, W2=W2out,
                             R1=R1, R2=R2)
    out = pl.pallas_call(
        body,
        out_shape=jax.ShapeDtypeStruct((N, R2, W2out), jnp.float32),
        grid_spec=pltpu.PrefetchScalarGridSpec(
            num_scalar_prefetch=1,
            grid=(N,),
            in_specs=[
                pl.BlockSpec((1, H, Wh), lambda n, sl: (n, 0, 0)),
                pl.BlockSpec((1, H, Wh), lambda n, sl: (n, 0, 0)),
                pl.BlockSpec((_KW1, R1, H), lambda n, sl: (0, 0, 0)),
                pl.BlockSpec((R1, 1), lambda n, sl: (0, 0)),
                pl.BlockSpec((_KW2, R2, R1), lambda n, sl: (0, 0, 0)),
                pl.BlockSpec((R2, 1), lambda n, sl: (0, 0)),
            ],
            out_specs=pl.BlockSpec((1, R2, W2out), lambda n, sl: (n, 0, 0)),
            scratch_shapes=[
                pltpu.VMEM((H, PW), jnp.bfloat16),
                pltpu.VMEM((H, PW), jnp.bfloat16),
                pltpu.VMEM((R1, W1out + 2 * _PW2), jnp.bfloat16),
            ],
        ),
        compiler_params=pltpu.CompilerParams(
            dimension_semantics=("parallel",),
            vmem_limit_bytes=_VMEM_LIMIT),
    )(seqs, xe, xo, band1, bias1, band2, bias2)
    return out.reshape(N, C2, Hout2, W2out), s2


# speedup vs baseline: 157.7231x; 1.1346x over previous
import functools

import jax
import jax.numpy as jnp
from jax import lax
from jax.experimental import pallas as pl
from jax.experimental.pallas import tpu as pltpu

_VMEM_LIMIT = 100 * 1024 * 1024

_KH1, _KW1, _SH1, _SW1, _PH1, _PW1 = 41, 11, 2, 2, 20, 5
_KH2, _KW2, _SH2, _SW2, _PH2, _PW2 = 21, 11, 2, 1, 10, 5


def _fused_body(seqs_ref, xe_ref, xo_ref, b1_ref, bias1_ref, b2_ref,
                bias2_ref, o_ref, *, B, H, Wh, W1, SEG, R1, R2):
    n = pl.program_id(0)
    WB = B * SEG

    oe, oo = (_PW1 + 1) // 2, (_PW1 - 1) // 2
    col = lax.broadcasted_iota(jnp.int32, (H, Wh), 1)
    pe_parts, po_parts = [], []
    for b in range(B):
        s0 = seqs_ref[n * B + b, 0]
        xe_b = jnp.where(2 * col < s0, jnp.clip(xe_ref[b], -1.0, 1.0),
                         jnp.bfloat16(0))
        xo_b = jnp.where(2 * col + 1 < s0, jnp.clip(xo_ref[b], -1.0, 1.0),
                         jnp.bfloat16(0))
        pe_parts.append(jnp.pad(xo_b, ((0, 0), (oe, SEG - oe - Wh))))
        po_parts.append(jnp.pad(xe_b, ((0, 0), (oo, SEG - oo - Wh))))
    pe = jnp.concatenate(pe_parts, axis=1)
    po = jnp.concatenate(po_parts, axis=1)

    x1_parts = []
    for kx in range(_KW1):
        plane = pe if kx % 2 == 0 else po
        j = kx // 2
        if j:
            plane = jnp.pad(plane[:, j:], ((0, 0), (0, j)))
        x1_parts.append(plane)
    x1 = jnp.concatenate(x1_parts, axis=0)
    acc1 = jnp.dot(b1_ref[...], x1, preferred_element_type=jnp.float32)
    y1 = jnp.clip(acc1 + bias1_ref[...], 0.0, 20.0)

    c_row = lax.broadcasted_iota(jnp.int32, (1, WB), 1)
    segidx = jnp.zeros((1, WB), jnp.int32)
    for b in range(1, B):
        segidx = segidx + (c_row >= b * SEG).astype(jnp.int32)
    wo_row = c_row - SEG * segidx
    thr1 = jnp.full((1, WB), seqs_ref[n * B, 1], jnp.int32)
    for b in range(1, B):
        thr1 = jnp.where(segidx == b, seqs_ref[n * B + b, 1], thr1)
    y1 = jnp.where(wo_row < thr1, y1, 0.0)

    y1x = jnp.pad(y1[:, :WB - _PW2], ((0, 0), (_PW2, 0)))
    y1x = y1x.astype(jnp.bfloat16)
    x2_parts = [y1x]
    for kx in range(1, _KW2):
        x2_parts.append(jnp.pad(y1x[:, kx:], ((0, 0), (0, kx))))
    x2 = jnp.concatenate(x2_parts, axis=0)
    acc2 = jnp.dot(b2_ref[...], x2, preferred_element_type=jnp.float32)
    y2 = jnp.clip(acc2 + bias2_ref[...], 0.0, 10.0)

    wcol = lax.broadcasted_iota(jnp.int32, (R2, W1), 1)
    for b in range(B):
        s2 = seqs_ref[n * B + b, 2]
        seg = y2[:, b * SEG:b * SEG + W1]
        o_ref[b] = jnp.where(wcol < s2, seg, 0.0)


def _sel(hout, hin, kh, stride, pad):
    ho = jnp.arange(hout)
    ky = jnp.arange(kh)
    h = jnp.arange(hin)
    return (h[None, None, :] ==
            (stride * ho[:, None, None] + ky[None, :, None] - pad)
            ).astype(jnp.float32)


def kernel(x, seq_lengths, w1, b1, w2, b2):
    N, Cin, H, W = x.shape
    C1 = w1.shape[0]
    C2 = w2.shape[0]
    Hout1 = (H + 2 * _PH1 - _KH1) // _SH1 + 1
    W1out = (W + 2 * _PW1 - _KW1) // _SW1 + 1
    Hout2 = (Hout1 + 2 * _PH2 - _KH2) // _SH2 + 1
    R1, R2 = C1 * Hout1, C2 * Hout2
    B = 4 if N % 4 == 0 else 1
    SEG = W1out + 16

    s0 = seq_lengths.astype(jnp.int32)
    s1 = (s0 + 2 * _PW1 - (_KW1 - 1) - 1) // _SW1 + 1
    s2 = (s1 + 2 * _PW2 - (_KW2 - 1) - 1) // _SW2 + 1
    seqs = jnp.stack([s0, s1, s2], axis=1)

    xsq = x[:, 0].astype(jnp.bfloat16)
    xe = xsq[:, :, 0::2]
    xo = xsq[:, :, 1::2]
    Wh = xe.shape[-1]

    sel1 = _sel(Hout1, H, _KH1, _SH1, _PH1)
    band1 = jnp.einsum('cyx,hyH->chxH', w1[:, 0], sel1)
    band1 = band1.reshape(R1, _KW1 * H).astype(jnp.bfloat16)
    sel2 = _sel(Hout2, Hout1, _KH2, _SH2, _PH2)
    band2 = jnp.einsum('oiyx,jyh->ojxih', w2, sel2)
    band2 = band2.reshape(R2, _KW2 * R1).astype(jnp.bfloat16)
    bias1 = jnp.repeat(b1, Hout1).reshape(R1, 1)
    bias2 = jnp.repeat(b2, Hout2).reshape(R2, 1)

    body = functools.partial(_fused_body, B=B, H=H, Wh=Wh, W1=W1out,
                             SEG=SEG, R1=R1, R2=R2)
    out = pl.pallas_call(
        body,
        out_shape=jax.ShapeDtypeStruct((N, R2, W1out), jnp.float32),
        grid_spec=pltpu.PrefetchScalarGridSpec(
            num_scalar_prefetch=1,
            grid=(N // B,),
            in_specs=[
                pl.BlockSpec((B, H, Wh), lambda n, sl: (n, 0, 0)),
                pl.BlockSpec((B, H, Wh), lambda n, sl: (n, 0, 0)),
                pl.BlockSpec((R1, _KW1 * H), lambda n, sl: (0, 0)),
                pl.BlockSpec((R1, 1), lambda n, sl: (0, 0)),
                pl.BlockSpec((R2, _KW2 * R1), lambda n, sl: (0, 0)),
                pl.BlockSpec((R2, 1), lambda n, sl: (0, 0)),
            ],
            out_specs=pl.BlockSpec((B, R2, W1out), lambda n, sl: (n, 0, 0)),
        ),
        compiler_params=pltpu.CompilerParams(
            dimension_semantics=("parallel",),
            vmem_limit_bytes=_VMEM_LIMIT),
    )(seqs, xe, xo, band1, bias1, band2, bias2)
    return out.reshape(N, C2, Hout2, W1out), s2
